# Initial kernel scaffold; baseline (speedup 1.0000x reference)
import jax
import jax.numpy as jnp
from jax.experimental import pallas as pl


def kernel(preds, targets, alpha):
    raise NotImplementedError("write your pallas kernel here")


# speedup vs baseline: 10.5359x; 10.5359x over previous
import jax
import jax.numpy as jnp
from jax.experimental import pallas as pl

GAMMA = 2.0
C = 150
HW = 224 * 224
PIX_BLOCK = 6272
N_PIX_BLOCKS = HW // PIX_BLOCK


def _focal_kernel(x_ref, t_ref, alpha_ref, acc_ref):
    n = pl.program_id(0)
    b = pl.program_id(1)

    @pl.when((n == 0) & (b == 0))
    def _():
        acc_ref[...] = jnp.zeros_like(acc_ref)

    x = x_ref[0]
    t = t_ref[0]
    alpha = alpha_ref[...]

    cls = jax.lax.broadcasted_iota(jnp.int32, x.shape, 0)
    mask = (cls == t).astype(jnp.float32)
    xt = jnp.sum(mask * x, axis=0)
    a = jnp.sum(mask * alpha, axis=0)

    m = jnp.max(x, axis=0)
    s = jnp.sum(jnp.exp(x - m), axis=0)
    log_pt = xt - m - jnp.log(s)
    pt = jnp.exp(log_pt)
    loss = -a * (1.0 - pt) ** GAMMA * log_pt
    acc_ref[...] += jnp.sum(loss).reshape(1, 1)


def kernel(preds, targets, alpha):
    N = preds.shape[0]
    x = preds.reshape(N, C, HW)
    t = targets.reshape(N, 1, HW).astype(jnp.int32)

    acc = pl.pallas_call(
        _focal_kernel,
        grid=(N, N_PIX_BLOCKS),
        in_specs=[
            pl.BlockSpec((1, C, PIX_BLOCK), lambda n, b: (n, 0, b)),
            pl.BlockSpec((1, 1, PIX_BLOCK), lambda n, b: (n, 0, b)),
            pl.BlockSpec((C, 1), lambda n, b: (0, 0)),
        ],
        out_specs=pl.BlockSpec((1, 1), lambda n, b: (0, 0)),
        out_shape=jax.ShapeDtypeStruct((1, 1), jnp.float32),
    )(x, t, alpha)

    return acc[0, 0] / (N * HW)
